# baseline (speedup 1.0000x reference)
import functools

import jax
import jax.numpy as jnp
from jax import lax
from jax.experimental import pallas as pl
from jax.experimental.pallas import tpu as pltpu
from jax.experimental.pallas import tpu_sc as plsc

_N_LEVELS = 10
_F = 2
_ROWS = 12 * ((2 ** (_N_LEVELS - 1)) ** 2 + 2)
_B = 16384
_OFFS = [(-1, -1), (-1, 0), (-1, 1), (0, -1), (0, 1), (1, -1), (1, 0), (1, 1)]

_NW = 32
_CHUNK = _B // _NW
_SUB = 64
_NSUB = _CHUNK // _SUB
_K = _N_LEVELS * 9
_GROWS = _K * _SUB // 128


def _prep_body(xt_ref, flat_ref, w_ref):
    theta = jnp.pi / 2.0 - jnp.deg2rad(xt_ref[0:1, :])
    phi = jnp.deg2rad(xt_ref[1:2, :])
    cos_t = jnp.cos(theta)
    for l in range(_N_LEVELS):
        nside = 2 ** l
        n_ring = 4 * nside
        n_col = 3 * nside
        ring = jnp.clip(jnp.floor(theta / jnp.pi * n_ring).astype(jnp.int32),
                        0, n_ring - 1)
        col = jnp.mod(jnp.floor(phi / (2.0 * jnp.pi) * n_col).astype(jnp.int32),
                      n_col)
        pix = ring * n_col + col
        flat_ref[l * 9:l * 9 + 1, :] = l * _ROWS + pix
        nr = jnp.concatenate([jnp.clip(ring + dr, 0, n_ring - 1)
                              for dr, _ in _OFFS], axis=0)
        nc = jnp.concatenate([jnp.mod(col + dc, n_col)
                              for _, dc in _OFFS], axis=0)
        npix = nr * n_col + nc
        flat_ref[l * 9 + 1:l * 9 + 9, :] = l * _ROWS + npix
        n_theta = (nr.astype(jnp.float32) + 0.5) / n_ring * jnp.pi
        n_phi = (nc.astype(jnp.float32) + 0.5) / n_col * 2.0 * jnp.pi
        dlon = n_phi - phi
        dlat = n_theta - theta
        a = (jnp.sin(dlat / 2.0) ** 2
             + cos_t * jnp.cos(n_theta) * jnp.sin(dlon / 2.0) ** 2)
        a = jnp.clip(a, 0.0, 1.0)
        dist = 2.0 * jnp.arctan2(jnp.sqrt(a), jnp.sqrt(1.0 - a))
        w_ref[l * 8:(l + 1) * 8, :] = dist / (jnp.sum(dist, axis=0,
                                                      keepdims=True) + 0.01)


def _prep(xt):
    return pl.pallas_call(
        _prep_body,
        grid=(_NW,),
        in_specs=[pl.BlockSpec((2, _CHUNK), lambda i: (0, i))],
        out_specs=[pl.BlockSpec((_K, _CHUNK), lambda i: (0, i)),
                   pl.BlockSpec((80, _CHUNK), lambda i: (0, i))],
        out_shape=[jax.ShapeDtypeStruct((_K, _B), jnp.int32),
                   jax.ShapeDtypeStruct((80, _B), jnp.float32)],
    )(xt)


def _sc_body(pixg_hbm, wt_hbm, tab_hbm, out_hbm, pix_v, rows_v, w_v, out_v, sem):
    wid = lax.axis_index("s") * 2 + lax.axis_index("c")
    iota = lax.iota(jnp.int32, 16)
    pvec = lax.shift_right_logical(iota, 1)
    fvec = lax.bitwise_and(iota, 1)
    sgen = pvec * (2 * _N_LEVELS) + fvec
    pltpu.sync_copy(wt_hbm.at[wid], w_v)

    def sub_body(s, carry):
        pltpu.sync_copy(pixg_hbm.at[wid, s], pix_v)
        copies = [
            pltpu.async_copy(tab_hbm.at[pix_v.at[k]],
                             rows_v.at[pl.ds(k * (2 * _SUB), 2 * _SUB)], sem)
            for k in range(_K)
        ]
        for c in copies:
            c.wait()

        def group_body(g, gcarry):
            goff = g * 16
            woff = s * (2 * _SUB) + goff
            accs = [jnp.zeros((16,), jnp.float32) for _ in range(_N_LEVELS)]
            for k in range(_K):
                l, j = divmod(k, 9)
                r16 = rows_v[pl.ds(k * (2 * _SUB) + goff, 16)]
                if j == 0:
                    accs[l] = accs[l] + r16
                else:
                    w16 = w_v[l * 8 + j - 1, pl.ds(woff, 16)]
                    accs[l] = accs[l] + w16 * r16
            sbase = (s * _SUB + g * 8) * (2 * _N_LEVELS) + sgen
            for l in range(_N_LEVELS):
                plsc.store_scatter(out_v, [sbase + 2 * l], accs[l])
            return gcarry

        lax.fori_loop(0, _SUB // 8, group_body, 0)
        return carry

    lax.fori_loop(0, _NSUB, sub_body, 0)
    pltpu.sync_copy(out_v,
                    out_hbm.at[pl.ds(wid * (_CHUNK * 2 * _N_LEVELS),
                                     _CHUNK * 2 * _N_LEVELS)])


def _sc_gather(pixg, wt, tab2):
    mesh = plsc.VectorSubcoreMesh(core_axis_name="c", subcore_axis_name="s")
    f = functools.partial(
        pl.kernel,
        out_type=jax.ShapeDtypeStruct((_B * 2 * _N_LEVELS,), jnp.float32),
        mesh=mesh,
        scratch_types=[
            pltpu.VMEM((_K, 128), jnp.int32),
            pltpu.VMEM((_K * 2 * _SUB,), jnp.float32),
            pltpu.VMEM((80, 2 * _CHUNK), jnp.float32),
            pltpu.VMEM((_CHUNK * 2 * _N_LEVELS,), jnp.float32),
            pltpu.SemaphoreType.DMA,
        ],
        compiler_params=pltpu.CompilerParams(needs_layout_passes=False),
    )(_sc_body)
    return f(pixg, wt, tab2)


def kernel(x, tables):
    xt = x.T
    flat, w = _prep(xt)
    flat2 = (2 * flat[:, :, None]
             + jnp.arange(2, dtype=jnp.int32)).reshape(_K, 2 * _B)
    pixg = (flat2.reshape(_K, _NW, _NSUB, 2 * _SUB)
                 .transpose(1, 2, 0, 3))
    w2 = jnp.repeat(w, 2, axis=1)
    wt = w2.reshape(80, _NW, 2 * _CHUNK).transpose(1, 0, 2)
    tabf = tables.reshape(_N_LEVELS * _ROWS * _F)
    out_flat = _sc_gather(pixg, wt, tabf)
    return out_flat.reshape(_B, 2 * _N_LEVELS)


# speedup vs baseline: 13.8816x; 13.8816x over previous
import functools

import jax
import jax.numpy as jnp
from jax import lax
from jax.experimental import pallas as pl
from jax.experimental.pallas import tpu as pltpu
from jax.experimental.pallas import tpu_sc as plsc

_N_LEVELS = 10
_F = 2
_ROWS = 12 * ((2 ** (_N_LEVELS - 1)) ** 2 + 2)
_B = 16384
_OFFS = [(-1, -1), (-1, 0), (-1, 1), (0, -1), (0, 1), (1, -1), (1, 0), (1, 1)]

_USED = [12 * 4 ** l for l in range(_N_LEVELS)]
_LOFF = [4 * (4 ** l - 1) for l in range(_N_LEVELS)]
_PLANE = sum(_USED)
_PLANE_PAD = _PLANE + 4

_NW = 32
_CHUNK = _B // _NW
_SUB = 64
_NSUB = _CHUNK // _SUB
_K = _N_LEVELS * 9
_NIDX = _K * _SUB


def _prep_body(xt_ref, w_ref):
    theta = jnp.pi / 2.0 - jnp.deg2rad(xt_ref[0:1, :])
    phi = jnp.deg2rad(xt_ref[1:2, :])
    cos_t = jnp.cos(theta)
    for l in range(_N_LEVELS):
        nside = 2 ** l
        n_ring = 4 * nside
        n_col = 3 * nside
        ring = jnp.clip(jnp.floor(theta / jnp.pi * n_ring).astype(jnp.int32),
                        0, n_ring - 1)
        col = jnp.mod(jnp.floor(phi / (2.0 * jnp.pi) * n_col).astype(jnp.int32),
                      n_col)
        nr = jnp.concatenate([jnp.clip(ring + dr, 0, n_ring - 1)
                              for dr, _ in _OFFS], axis=0)
        nc = jnp.concatenate([jnp.mod(col + dc, n_col)
                              for _, dc in _OFFS], axis=0)
        n_theta = (nr.astype(jnp.float32) + 0.5) / n_ring * jnp.pi
        n_phi = (nc.astype(jnp.float32) + 0.5) / n_col * 2.0 * jnp.pi
        dlon = n_phi - phi
        dlat = n_theta - theta
        a = (jnp.sin(dlat / 2.0) ** 2
             + cos_t * jnp.cos(n_theta) * jnp.sin(dlon / 2.0) ** 2)
        a = jnp.clip(a, 0.0, 1.0)
        dist = 2.0 * jnp.arctan2(jnp.sqrt(a), jnp.sqrt(1.0 - a))
        w_ref[l * 8:(l + 1) * 8, :] = dist / (jnp.sum(dist, axis=0,
                                                      keepdims=True) + 0.01)


def _prep(xt):
    return pl.pallas_call(
        _prep_body,
        grid=(_NW,),
        in_specs=[pl.BlockSpec((2, _CHUNK), lambda i: (0, i))],
        out_specs=pl.BlockSpec((80, _CHUNK), lambda i: (0, i)),
        out_shape=jax.ShapeDtypeStruct((80, _B), jnp.float32),
    )(xt)


def _indices(x):
    rad = jnp.deg2rad(x)
    theta = jnp.pi / 2.0 - rad[:, 0]
    phi = rad[:, 1]
    rows = []
    for l in range(_N_LEVELS):
        nside = 2 ** l
        n_ring = 4 * nside
        n_col = 3 * nside
        ring = jnp.clip(jnp.floor(theta / jnp.pi * n_ring).astype(jnp.int32),
                        0, n_ring - 1)
        col = jnp.mod(jnp.floor(phi / (2.0 * jnp.pi) * n_col).astype(jnp.int32),
                      n_col)
        rows.append(_LOFF[l] + ring * n_col + col)
        for dr, dc in _OFFS:
            nr = jnp.clip(ring + dr, 0, n_ring - 1)
            nc = jnp.mod(col + dc, n_col)
            rows.append(_LOFF[l] + nr * n_col + nc)
    return jnp.stack(rows, axis=0)


def _sc_body(pixg_hbm, wt_hbm, tab_hbm, out_hbm, pix_v, rows0_v, rows1_v,
             w_v, out_v, sem):
    wid = lax.axis_index("s") * 2 + lax.axis_index("c")
    tabf1 = tab_hbm.at[pl.ds(_PLANE_PAD, _PLANE)]
    pltpu.sync_copy(wt_hbm.at[wid], w_v)

    def sub_body(s, carry):
        pltpu.sync_copy(pixg_hbm.at[wid, s], pix_v)
        c0 = pltpu.async_copy(tab_hbm.at[pix_v], rows0_v, sem)
        c1 = pltpu.async_copy(tabf1.at[pix_v], rows1_v, sem)
        c0.wait()
        c1.wait()

        def group_body(g, gcarry):
            woff = s * _SUB + g * 16
            for k in range(_K):
                l, j = divmod(k, 9)
                f0 = rows0_v[pl.ds(k * _SUB + g * 16, 16)]
                f1 = rows1_v[pl.ds(k * _SUB + g * 16, 16)]
                if j == 0:
                    acc0, acc1 = f0, f1
                else:
                    w16 = w_v[l * 8 + j - 1, pl.ds(woff, 16)]
                    acc0 = acc0 + w16 * f0
                    acc1 = acc1 + w16 * f1
                if j == 8:
                    out_v[2 * l, pl.ds(woff, 16)] = acc0
                    out_v[2 * l + 1, pl.ds(woff, 16)] = acc1
            return gcarry

        lax.fori_loop(0, _SUB // 16, group_body, 0)
        return carry

    lax.fori_loop(0, _NSUB, sub_body, 0)
    pltpu.sync_copy(out_v, out_hbm.at[:, pl.ds(wid * _CHUNK, _CHUNK)])


def _sc_gather(pixg, wt, tab):
    mesh = plsc.VectorSubcoreMesh(core_axis_name="c", subcore_axis_name="s")
    f = functools.partial(
        pl.kernel,
        out_type=jax.ShapeDtypeStruct((2 * _N_LEVELS, _B), jnp.float32),
        mesh=mesh,
        scratch_types=[
            pltpu.VMEM((_NIDX,), jnp.int32),
            pltpu.VMEM((_NIDX,), jnp.float32),
            pltpu.VMEM((_NIDX,), jnp.float32),
            pltpu.VMEM((80, _CHUNK), jnp.float32),
            pltpu.VMEM((2 * _N_LEVELS, _CHUNK), jnp.float32),
            pltpu.SemaphoreType.DMA,
        ],
        compiler_params=pltpu.CompilerParams(needs_layout_passes=False),
    )(_sc_body)
    return f(pixg, wt, tab)


def kernel(x, tables):
    xt = x.T
    w = _prep(xt)
    wordidx = _indices(x)
    pixg = (wordidx.reshape(_K, _NW, _NSUB, _SUB)
                   .transpose(1, 2, 0, 3)
                   .reshape(_NW, _NSUB, _NIDX))
    wt = w.reshape(80, _NW, _CHUNK).transpose(1, 0, 2)
    f0s = [tables[l, :_USED[l], 0] for l in range(_N_LEVELS)]
    f1s = [tables[l, :_USED[l], 1] for l in range(_N_LEVELS)]
    packed = jnp.concatenate(
        f0s + [jnp.zeros((4,), jnp.float32)] + f1s)
    out2d = _sc_gather(pixg, wt, packed)
    return out2d.T


# speedup vs baseline: 14.3073x; 1.0307x over previous
import functools

import jax
import jax.numpy as jnp
from jax import lax
from jax.experimental import pallas as pl
from jax.experimental.pallas import tpu as pltpu
from jax.experimental.pallas import tpu_sc as plsc

_N_LEVELS = 10
_F = 2
_ROWS = 12 * ((2 ** (_N_LEVELS - 1)) ** 2 + 2)
_B = 16384
_OFFS = [(-1, -1), (-1, 0), (-1, 1), (0, -1), (0, 1), (1, -1), (1, 0), (1, 1)]

_USED = [12 * 4 ** l for l in range(_N_LEVELS)]
_LOFF = [4 * (4 ** l - 1) for l in range(_N_LEVELS)]
_PLANE = sum(_USED)
_PLANE_PAD = _PLANE + 4

_NW = 32
_CHUNK = _B // _NW
_SUB = 64
_NSUB = _CHUNK // _SUB
_K = _N_LEVELS * 9
_NIDX = _K * _SUB
_NSPLIT = 4


def _prep_body(xt_ref, w_ref):
    theta = jnp.pi / 2.0 - jnp.deg2rad(xt_ref[0:1, :])
    phi = jnp.deg2rad(xt_ref[1:2, :])
    cos_t = jnp.cos(theta)
    for l in range(_N_LEVELS):
        nside = 2 ** l
        n_ring = 4 * nside
        n_col = 3 * nside
        ring = jnp.clip(jnp.floor(theta / jnp.pi * n_ring).astype(jnp.int32),
                        0, n_ring - 1)
        col = jnp.mod(jnp.floor(phi / (2.0 * jnp.pi) * n_col).astype(jnp.int32),
                      n_col)
        nr = jnp.concatenate([jnp.clip(ring + dr, 0, n_ring - 1)
                              for dr, _ in _OFFS], axis=0)
        nc = jnp.concatenate([jnp.mod(col + dc, n_col)
                              for _, dc in _OFFS], axis=0)
        n_theta = (nr.astype(jnp.float32) + 0.5) / n_ring * jnp.pi
        n_phi = (nc.astype(jnp.float32) + 0.5) / n_col * 2.0 * jnp.pi
        dlon = n_phi - phi
        dlat = n_theta - theta
        a = (jnp.sin(dlat / 2.0) ** 2
             + cos_t * jnp.cos(n_theta) * jnp.sin(dlon / 2.0) ** 2)
        a = jnp.clip(a, 0.0, 1.0)
        dist = 2.0 * jnp.arctan2(jnp.sqrt(a), jnp.sqrt(1.0 - a))
        w_ref[l * 8:(l + 1) * 8, :] = dist / (jnp.sum(dist, axis=0,
                                                      keepdims=True) + 0.01)


def _prep(xt):
    return pl.pallas_call(
        _prep_body,
        grid=(_NW,),
        in_specs=[pl.BlockSpec((2, _CHUNK), lambda i: (0, i))],
        out_specs=pl.BlockSpec((80, _CHUNK), lambda i: (0, i)),
        out_shape=jax.ShapeDtypeStruct((80, _B), jnp.float32),
    )(xt)


def _indices(x):
    rad = jnp.deg2rad(x)
    theta = jnp.pi / 2.0 - rad[:, 0]
    phi = rad[:, 1]
    rows = []
    for l in range(_N_LEVELS):
        nside = 2 ** l
        n_ring = 4 * nside
        n_col = 3 * nside
        ring = jnp.clip(jnp.floor(theta / jnp.pi * n_ring).astype(jnp.int32),
                        0, n_ring - 1)
        col = jnp.mod(jnp.floor(phi / (2.0 * jnp.pi) * n_col).astype(jnp.int32),
                      n_col)
        rows.append(_LOFF[l] + ring * n_col + col)
        for dr, dc in _OFFS:
            nr = jnp.clip(ring + dr, 0, n_ring - 1)
            nc = jnp.mod(col + dc, n_col)
            rows.append(_LOFF[l] + nr * n_col + nc)
    return jnp.stack(rows, axis=0)


def _sc_body(pixg_hbm, wt_hbm, tab_hbm, out_hbm, pix_v, rows0_v, rows1_v,
             w_v, out_v, sem):
    wid = lax.axis_index("s") * 2 + lax.axis_index("c")
    tabf1 = tab_hbm.at[pl.ds(_PLANE_PAD, _PLANE)]
    pltpu.sync_copy(wt_hbm.at[wid], w_v)

    def sub_body(s, carry):
        pltpu.sync_copy(pixg_hbm.at[wid, s], pix_v)
        cs = []
        for j in range(_NSPLIT):
            sl = pl.ds(j * (_NIDX // _NSPLIT), _NIDX // _NSPLIT)
            cs.append(pltpu.async_copy(tab_hbm.at[pix_v.at[sl]],
                                       rows0_v.at[sl], sem))
            cs.append(pltpu.async_copy(tabf1.at[pix_v.at[sl]],
                                       rows1_v.at[sl], sem))
        for c in cs:
            c.wait()

        def group_body(g, gcarry):
            woff = s * _SUB + g * 16
            for k in range(_K):
                l, j = divmod(k, 9)
                f0 = rows0_v[pl.ds(k * _SUB + g * 16, 16)]
                f1 = rows1_v[pl.ds(k * _SUB + g * 16, 16)]
                if j == 0:
                    acc0, acc1 = f0, f1
                else:
                    w16 = w_v[l * 8 + j - 1, pl.ds(woff, 16)]
                    acc0 = acc0 + w16 * f0
                    acc1 = acc1 + w16 * f1
                if j == 8:
                    out_v[2 * l, pl.ds(woff, 16)] = acc0
                    out_v[2 * l + 1, pl.ds(woff, 16)] = acc1
            return gcarry

        lax.fori_loop(0, _SUB // 16, group_body, 0)
        return carry

    lax.fori_loop(0, _NSUB, sub_body, 0)
    pltpu.sync_copy(out_v, out_hbm.at[:, pl.ds(wid * _CHUNK, _CHUNK)])


def _sc_gather(pixg, wt, tab):
    mesh = plsc.VectorSubcoreMesh(core_axis_name="c", subcore_axis_name="s")
    f = functools.partial(
        pl.kernel,
        out_type=jax.ShapeDtypeStruct((2 * _N_LEVELS, _B), jnp.float32),
        mesh=mesh,
        scratch_types=[
            pltpu.VMEM((_NIDX,), jnp.int32),
            pltpu.VMEM((_NIDX,), jnp.float32),
            pltpu.VMEM((_NIDX,), jnp.float32),
            pltpu.VMEM((80, _CHUNK), jnp.float32),
            pltpu.VMEM((2 * _N_LEVELS, _CHUNK), jnp.float32),
            pltpu.SemaphoreType.DMA,
        ],
        compiler_params=pltpu.CompilerParams(needs_layout_passes=False),
    )(_sc_body)
    return f(pixg, wt, tab)


def kernel(x, tables):
    xt = x.T
    w = _prep(xt)
    wordidx = _indices(x)
    pixg = (wordidx.reshape(_K, _NW, _NSUB, _SUB)
                   .transpose(1, 2, 0, 3)
                   .reshape(_NW, _NSUB, _NIDX))
    wt = w.reshape(80, _NW, _CHUNK).transpose(1, 0, 2)
    f0s = [tables[l, :_USED[l], 0] for l in range(_N_LEVELS)]
    f1s = [tables[l, :_USED[l], 1] for l in range(_N_LEVELS)]
    packed = jnp.concatenate(
        f0s + [jnp.zeros((4,), jnp.float32)] + f1s)
    out2d = _sc_gather(pixg, wt, packed)
    return out2d.T


# speedup vs baseline: 55.4715x; 3.8771x over previous
import functools

import jax
import jax.numpy as jnp
from jax import lax
from jax.experimental import pallas as pl
from jax.experimental.pallas import tpu as pltpu
from jax.experimental.pallas import tpu_sc as plsc

_N_LEVELS = 10
_F = 2
_ROWS = 12 * ((2 ** (_N_LEVELS - 1)) ** 2 + 2)
_B = 16384
_OFFS = [(-1, -1), (-1, 0), (-1, 1), (0, -1), (0, 1), (1, -1), (1, 0), (1, 1)]

_USED = [12 * 4 ** l for l in range(_N_LEVELS)]
_LOFF = [0] + [4 * (4 ** l - 1) + 4 for l in range(1, _N_LEVELS)]
_PLANE = _LOFF[-1] + _USED[-1]
_PLANE_PAD = _PLANE

_NW = 32
_CHUNK = _B // _NW
_SUB = 64
_NSUB = _CHUNK // _SUB
_K = _N_LEVELS * 9
_NIDX = _K * _SUB
_L6 = _LOFF[6]
_L9 = _LOFF[9]
_WB = 1792
_WC = 640


def _prep_body(xt_ref, w_ref):
    theta = jnp.pi / 2.0 - jnp.deg2rad(xt_ref[0:1, :])
    phi = jnp.deg2rad(xt_ref[1:2, :])
    cos_t = jnp.cos(theta)
    for l in range(_N_LEVELS):
        nside = 2 ** l
        n_ring = 4 * nside
        n_col = 3 * nside
        ring = jnp.clip(jnp.floor(theta / jnp.pi * n_ring).astype(jnp.int32),
                        0, n_ring - 1)
        col = jnp.mod(jnp.floor(phi / (2.0 * jnp.pi) * n_col).astype(jnp.int32),
                      n_col)
        nr = jnp.concatenate([jnp.clip(ring + dr, 0, n_ring - 1)
                              for dr, _ in _OFFS], axis=0)
        nc = jnp.concatenate([jnp.mod(col + dc, n_col)
                              for _, dc in _OFFS], axis=0)
        n_theta = (nr.astype(jnp.float32) + 0.5) / n_ring * jnp.pi
        n_phi = (nc.astype(jnp.float32) + 0.5) / n_col * 2.0 * jnp.pi
        dlon = n_phi - phi
        dlat = n_theta - theta
        a = (jnp.sin(dlat / 2.0) ** 2
             + cos_t * jnp.cos(n_theta) * jnp.sin(dlon / 2.0) ** 2)
        a = jnp.clip(a, 0.0, 1.0)
        dist = 2.0 * jnp.arctan2(jnp.sqrt(a), jnp.sqrt(1.0 - a))
        w_ref[l * 8:(l + 1) * 8, :] = dist / (jnp.sum(dist, axis=0,
                                                      keepdims=True) + 0.01)


def _prep(xt):
    return pl.pallas_call(
        _prep_body,
        grid=(_NW,),
        in_specs=[pl.BlockSpec((2, _CHUNK), lambda i: (0, i))],
        out_specs=pl.BlockSpec((80, _CHUNK), lambda i: (0, i)),
        out_shape=jax.ShapeDtypeStruct((80, _B), jnp.float32),
    )(xt)


def _indices(x):
    rad = jnp.deg2rad(x)
    theta = jnp.pi / 2.0 - rad[:, 0]
    phi = rad[:, 1]
    rows = []
    for l in range(_N_LEVELS):
        nside = 2 ** l
        n_ring = 4 * nside
        n_col = 3 * nside
        ring = jnp.clip(jnp.floor(theta / jnp.pi * n_ring).astype(jnp.int32),
                        0, n_ring - 1)
        col = jnp.mod(jnp.floor(phi / (2.0 * jnp.pi) * n_col).astype(jnp.int32),
                      n_col)
        rows.append(_LOFF[l] + ring * n_col + col)
        for dr, dc in _OFFS:
            nr = jnp.clip(ring + dr, 0, n_ring - 1)
            nc = jnp.mod(col + dc, n_col)
            rows.append(_LOFF[l] + nr * n_col + nc)
    return jnp.stack(rows, axis=0)


def _sc_body(pixa_hbm, pixb_hbm, pixc_hbm, wt_hbm, tab_hbm, out_hbm,
             s0f0_v, s0f1_v, pixa_v, pixb_v, pixc_v,
             rb0_v, rb1_v, rc0_v, rc1_v, w_v, out_v, semb, semc):
    cid = lax.axis_index("c")
    sid = lax.axis_index("s")
    wid = sid * 2 + cid
    tabf1 = tab_hbm.at[pl.ds(_PLANE_PAD, _PLANE)]

    pltpu.sync_copy(pixc_hbm.at[wid], pixc_v)
    ccs = []
    for s8 in range(_NSUB):
        sl = pl.ds(s8 * _WC, _WC)
        ccs.append(pltpu.async_copy(tab_hbm.at[pixc_v.at[sl]],
                                    rc0_v.at[sl], semc))
        ccs.append(pltpu.async_copy(tabf1.at[pixc_v.at[sl]],
                                    rc1_v.at[sl], semc))

    pltpu.sync_copy(tab_hbm.at[pl.ds(0, _L6)], s0f0_v)
    pltpu.sync_copy(tab_hbm.at[pl.ds(_PLANE_PAD, _L6)], s0f1_v)

    pltpu.sync_copy(pixb_hbm.at[wid], pixb_v)

    cbs = []
    for s8 in range(_NSUB):
        sl = pl.ds(s8 * _WB, _WB)
        cbs.append(pltpu.async_copy(tab_hbm.at[pixb_v.at[sl]],
                                    rb0_v.at[sl], semb))
        cbs.append(pltpu.async_copy(tabf1.at[pixb_v.at[sl]],
                                    rb1_v.at[sl], semb))
    for c in ccs:
        c.wait()
    for c in cbs:
        c.wait()

    def sub_body(s, carry):
        pltpu.sync_copy(pixa_hbm.at[wid, s], pixa_v)
        pltpu.sync_copy(wt_hbm.at[wid, s], w_v)

        def group_body(g, gcarry):
            woff = s * _SUB + g * 16
            for k in range(_K):
                l, j = divmod(k, 9)
                if k < 54:
                    idx16 = pixa_v[pl.ds(k * _SUB + g * 16, 16)]
                    f0 = plsc.load_gather(s0f0_v, [idx16])
                    f1 = plsc.load_gather(s0f1_v, [idx16])
                elif k < 81:
                    ob = pl.ds(s * _WB + (k - 54) * _SUB + g * 16, 16)
                    f0 = rb0_v[ob]
                    f1 = rb1_v[ob]
                else:
                    oc = pl.ds(s * _WC + (k - 81) * _SUB + g * 16, 16)
                    f0 = rc0_v[oc]
                    f1 = rc1_v[oc]
                if j == 0:
                    acc0, acc1 = f0, f1
                else:
                    w16 = w_v[l * 8 + j - 1, pl.ds(g * 16, 16)]
                    acc0 = acc0 + w16 * f0
                    acc1 = acc1 + w16 * f1
                if j == 8:
                    out_v[2 * l, pl.ds(woff, 16)] = acc0
                    out_v[2 * l + 1, pl.ds(woff, 16)] = acc1
            return gcarry

        lax.fori_loop(0, _SUB // 16, group_body, 0)
        return carry

    lax.fori_loop(0, _NSUB, sub_body, 0)
    pltpu.sync_copy(out_v, out_hbm.at[:, pl.ds(wid * _CHUNK, _CHUNK)])


def _sc_gather(pixa, pixb, pixc, wt, tab):
    mesh = plsc.VectorSubcoreMesh(core_axis_name="c", subcore_axis_name="s")
    f = functools.partial(
        pl.kernel,
        out_type=jax.ShapeDtypeStruct((2 * _N_LEVELS, _B), jnp.float32),
        mesh=mesh,
        scratch_types=[
            pltpu.VMEM((_L6,), jnp.float32),
            pltpu.VMEM((_L6,), jnp.float32),
            pltpu.VMEM((54 * _SUB,), jnp.int32),
            pltpu.VMEM((_NSUB * _WB,), jnp.int32),
            pltpu.VMEM((_NSUB * _WC,), jnp.int32),
            pltpu.VMEM((_NSUB * _WB,), jnp.float32),
            pltpu.VMEM((_NSUB * _WB,), jnp.float32),
            pltpu.VMEM((_NSUB * _WC,), jnp.float32),
            pltpu.VMEM((_NSUB * _WC,), jnp.float32),
            pltpu.VMEM((80, _SUB), jnp.float32),
            pltpu.VMEM((2 * _N_LEVELS, _CHUNK), jnp.float32),
            pltpu.SemaphoreType.DMA,
            pltpu.SemaphoreType.DMA,
        ],
        compiler_params=pltpu.CompilerParams(needs_layout_passes=False),
    )(_sc_body)
    return f(pixa, pixb, pixc, wt, tab)


def kernel(x, tables):
    xt = x.T
    w = _prep(xt)
    wordidx = _indices(x)
    widx_t = (wordidx.reshape(_K, _NW, _NSUB, _SUB)
                     .transpose(1, 2, 0, 3))
    pixa = widx_t[:, :, :54, :].reshape(_NW, _NSUB, 54 * _SUB)
    padb = jnp.zeros((_NW, _NSUB, _WB - 27 * _SUB), jnp.int32)
    padc = jnp.zeros((_NW, _NSUB, _WC - 9 * _SUB), jnp.int32)
    pixb = jnp.concatenate(
        [widx_t[:, :, 54:81, :].reshape(_NW, _NSUB, 27 * _SUB), padb],
        axis=2).reshape(_NW, _NSUB * _WB)
    pixc = jnp.concatenate(
        [widx_t[:, :, 81:, :].reshape(_NW, _NSUB, 9 * _SUB), padc],
        axis=2).reshape(_NW, _NSUB * _WC)
    wt = w.reshape(80, _NW, _NSUB, _SUB).transpose(1, 2, 0, 3)
    pad4 = jnp.zeros((4,), jnp.float32)
    f0s = [tables[0, :_USED[0], 0], pad4] + [tables[l, :_USED[l], 0]
                                             for l in range(1, _N_LEVELS)]
    f1s = [tables[0, :_USED[0], 1], pad4] + [tables[l, :_USED[l], 1]
                                             for l in range(1, _N_LEVELS)]
    packed = jnp.concatenate(f0s + f1s)
    out2d = _sc_gather(pixa, pixb, pixc, wt, packed)
    return out2d.T


# speedup vs baseline: 58.3674x; 1.0522x over previous
import functools

import jax
import jax.numpy as jnp
from jax import lax
from jax.experimental import pallas as pl
from jax.experimental.pallas import tpu as pltpu
from jax.experimental.pallas import tpu_sc as plsc

_N_LEVELS = 10
_F = 2
_ROWS = 12 * ((2 ** (_N_LEVELS - 1)) ** 2 + 2)
_B = 16384
_OFFS = [(-1, -1), (-1, 0), (-1, 1), (0, -1), (0, 1), (1, -1), (1, 0), (1, 1)]

_USED = [12 * 4 ** l for l in range(_N_LEVELS)]
_LOFF = [0] + [4 * (4 ** l - 1) + 4 for l in range(1, _N_LEVELS)]
_PLANE = _LOFF[-1] + _USED[-1]
_PLANE_PAD = _PLANE

_NW = 32
_CHUNK = _B // _NW
_SUB = 64
_NSUB = _CHUNK // _SUB
_K = _N_LEVELS * 9
_NIDX = _K * _SUB
_L6 = _LOFF[6]
_L9 = _LOFF[9]
_WB = 1792
_WC = 640


def _prep_body(xt_ref, w_ref):
    theta = jnp.pi / 2.0 - jnp.deg2rad(xt_ref[0:1, :])
    phi = jnp.deg2rad(xt_ref[1:2, :])
    cos_t = jnp.cos(theta)
    for l in range(_N_LEVELS):
        nside = 2 ** l
        n_ring = 4 * nside
        n_col = 3 * nside
        ring = jnp.clip(jnp.floor(theta / jnp.pi * n_ring).astype(jnp.int32),
                        0, n_ring - 1)
        col = jnp.mod(jnp.floor(phi / (2.0 * jnp.pi) * n_col).astype(jnp.int32),
                      n_col)
        nr = jnp.concatenate([jnp.clip(ring + dr, 0, n_ring - 1)
                              for dr, _ in _OFFS], axis=0)
        nc = jnp.concatenate([jnp.mod(col + dc, n_col)
                              for _, dc in _OFFS], axis=0)
        n_theta = (nr.astype(jnp.float32) + 0.5) / n_ring * jnp.pi
        n_phi = (nc.astype(jnp.float32) + 0.5) / n_col * 2.0 * jnp.pi
        dlon = n_phi - phi
        dlat = n_theta - theta
        a = (jnp.sin(dlat / 2.0) ** 2
             + cos_t * jnp.cos(n_theta) * jnp.sin(dlon / 2.0) ** 2)
        a = jnp.clip(a, 0.0, 1.0)
        dist = 2.0 * jnp.arctan2(jnp.sqrt(a), jnp.sqrt(1.0 - a))
        wgt = dist / (jnp.sum(dist, axis=0, keepdims=True) + 0.01)
        for s in range(_NSUB):
            w_ref[0, s, l * 8:(l + 1) * 8, :] = wgt[:, s * _SUB:(s + 1) * _SUB]


def _prep(xt):
    return pl.pallas_call(
        _prep_body,
        grid=(_NW,),
        in_specs=[pl.BlockSpec((2, _CHUNK), lambda i: (0, i))],
        out_specs=pl.BlockSpec((1, _NSUB, 80, _SUB), lambda i: (i, 0, 0, 0)),
        out_shape=jax.ShapeDtypeStruct((_NW, _NSUB, 80, _SUB), jnp.float32),
    )(xt)


def _indices(x):
    rad = jnp.deg2rad(x)
    theta = jnp.pi / 2.0 - rad[:, 0]
    phi = rad[:, 1]
    tiers = {"a": [], "b": [], "c": []}
    for l in range(_N_LEVELS):
        nside = 2 ** l
        n_ring = 4 * nside
        n_col = 3 * nside
        dst = tiers["a"] if l < 6 else (tiers["b"] if l < 9 else tiers["c"])
        ring = jnp.clip(jnp.floor(theta / jnp.pi * n_ring).astype(jnp.int32),
                        0, n_ring - 1)
        col = jnp.mod(jnp.floor(phi / (2.0 * jnp.pi) * n_col).astype(jnp.int32),
                      n_col)
        dst.append(_LOFF[l] + ring * n_col + col)
        for dr, dc in _OFFS:
            nr = jnp.clip(ring + dr, 0, n_ring - 1)
            nc = jnp.mod(col + dc, n_col)
            dst.append(_LOFF[l] + nr * n_col + nc)
    dummy = jnp.zeros((_B,), jnp.int32)
    tiers["b"].append(dummy)
    tiers["c"].append(dummy)

    def fmt(rows):
        nk = len(rows)
        return (jnp.stack(rows, axis=0)
                   .reshape(nk, _NW, _NSUB, _SUB)
                   .transpose(1, 2, 0, 3)
                   .reshape(_NW, _NSUB * nk * _SUB))

    return fmt(tiers["a"]), fmt(tiers["b"]), fmt(tiers["c"])


def _sc_body(pixa_hbm, pixb_hbm, pixc_hbm, wt_hbm, tab_hbm, out_hbm,
             s0f0_v, s0f1_v, pixa_v, pixb_v, pixc_v,
             rb0_v, rb1_v, rc0_v, rc1_v, w_v, out_v, semb, semc):
    cid = lax.axis_index("c")
    sid = lax.axis_index("s")
    wid = sid * 2 + cid
    tabf1 = tab_hbm.at[pl.ds(_PLANE_PAD, _PLANE)]

    pltpu.sync_copy(pixc_hbm.at[wid], pixc_v)
    ccs = []
    for s8 in range(_NSUB):
        sl = pl.ds(s8 * _WC, _WC)
        ccs.append(pltpu.async_copy(tab_hbm.at[pixc_v.at[sl]],
                                    rc0_v.at[sl], semc))
        ccs.append(pltpu.async_copy(tabf1.at[pixc_v.at[sl]],
                                    rc1_v.at[sl], semc))

    pltpu.sync_copy(tab_hbm.at[pl.ds(0, _L6)], s0f0_v)
    pltpu.sync_copy(tab_hbm.at[pl.ds(_PLANE_PAD, _L6)], s0f1_v)

    pltpu.sync_copy(pixb_hbm.at[wid], pixb_v)

    cbs = []
    for s8 in range(_NSUB):
        sl = pl.ds(s8 * _WB, _WB)
        cbs.append(pltpu.async_copy(tab_hbm.at[pixb_v.at[sl]],
                                    rb0_v.at[sl], semb))
        cbs.append(pltpu.async_copy(tabf1.at[pixb_v.at[sl]],
                                    rb1_v.at[sl], semb))
    for c in ccs:
        c.wait()
    for c in cbs:
        c.wait()

    def sub_body(s, carry):
        pltpu.sync_copy(pixa_hbm.at[wid, pl.ds(s * 54 * _SUB, 54 * _SUB)],
                        pixa_v)
        pltpu.sync_copy(wt_hbm.at[wid, s], w_v)

        def group_body(g, gcarry):
            woff = s * _SUB + g * 16
            for k in range(_K):
                l, j = divmod(k, 9)
                if k < 54:
                    idx16 = pixa_v[pl.ds(k * _SUB + g * 16, 16)]
                    f0 = plsc.load_gather(s0f0_v, [idx16])
                    f1 = plsc.load_gather(s0f1_v, [idx16])
                elif k < 81:
                    ob = pl.ds(s * _WB + (k - 54) * _SUB + g * 16, 16)
                    f0 = rb0_v[ob]
                    f1 = rb1_v[ob]
                else:
                    oc = pl.ds(s * _WC + (k - 81) * _SUB + g * 16, 16)
                    f0 = rc0_v[oc]
                    f1 = rc1_v[oc]
                if j == 0:
                    acc0, acc1 = f0, f1
                else:
                    w16 = w_v[l * 8 + j - 1, pl.ds(g * 16, 16)]
                    acc0 = acc0 + w16 * f0
                    acc1 = acc1 + w16 * f1
                if j == 8:
                    out_v[2 * l, pl.ds(woff, 16)] = acc0
                    out_v[2 * l + 1, pl.ds(woff, 16)] = acc1
            return gcarry

        lax.fori_loop(0, _SUB // 16, group_body, 0)
        return carry

    lax.fori_loop(0, _NSUB, sub_body, 0)
    pltpu.sync_copy(out_v, out_hbm.at[:, pl.ds(wid * _CHUNK, _CHUNK)])


def _sc_gather(pixa, pixb, pixc, wt, tab):
    mesh = plsc.VectorSubcoreMesh(core_axis_name="c", subcore_axis_name="s")
    f = functools.partial(
        pl.kernel,
        out_type=jax.ShapeDtypeStruct((2 * _N_LEVELS, _B), jnp.float32),
        mesh=mesh,
        scratch_types=[
            pltpu.VMEM((_L6,), jnp.float32),
            pltpu.VMEM((_L6,), jnp.float32),
            pltpu.VMEM((54 * _SUB,), jnp.int32),
            pltpu.VMEM((_NSUB * _WB,), jnp.int32),
            pltpu.VMEM((_NSUB * _WC,), jnp.int32),
            pltpu.VMEM((_NSUB * _WB,), jnp.float32),
            pltpu.VMEM((_NSUB * _WB,), jnp.float32),
            pltpu.VMEM((_NSUB * _WC,), jnp.float32),
            pltpu.VMEM((_NSUB * _WC,), jnp.float32),
            pltpu.VMEM((80, _SUB), jnp.float32),
            pltpu.VMEM((2 * _N_LEVELS, _CHUNK), jnp.float32),
            pltpu.SemaphoreType.DMA,
            pltpu.SemaphoreType.DMA,
        ],
        compiler_params=pltpu.CompilerParams(needs_layout_passes=False),
    )(_sc_body)
    return f(pixa, pixb, pixc, wt, tab)


def kernel(x, tables):
    xt = x.T
    wt = _prep(xt)
    pixa, pixb, pixc = _indices(x)
    pad4 = jnp.zeros((4,), jnp.float32)
    f0s = [tables[0, :_USED[0], 0], pad4] + [tables[l, :_USED[l], 0]
                                             for l in range(1, _N_LEVELS)]
    f1s = [tables[0, :_USED[0], 1], pad4] + [tables[l, :_USED[l], 1]
                                             for l in range(1, _N_LEVELS)]
    packed = jnp.concatenate(f0s + f1s)
    out2d = _sc_gather(pixa, pixb, pixc, wt, packed)
    return out2d.T


# speedup vs baseline: 59.7048x; 1.0229x over previous
import functools

import jax
import jax.numpy as jnp
from jax import lax
from jax.experimental import pallas as pl
from jax.experimental.pallas import tpu as pltpu
from jax.experimental.pallas import tpu_sc as plsc

_N_LEVELS = 10
_F = 2
_ROWS = 12 * ((2 ** (_N_LEVELS - 1)) ** 2 + 2)
_B = 16384
_OFFS = [(-1, -1), (-1, 0), (-1, 1), (0, -1), (0, 1), (1, -1), (1, 0), (1, 1)]

_USED = [12 * 4 ** l for l in range(_N_LEVELS)]
_TILES = [-(-u // 128) for u in _USED]
_NOFF = [0]
for _tl in _TILES:
    _NOFF.append(_NOFF[-1] + _tl * 256)
_TOT = _NOFF[_N_LEVELS]

_NW = 32
_CHUNK = _B // _NW
_SUB = 64
_NSUB = _CHUNK // _SUB
_K = _N_LEVELS * 9
_NIDX = _K * _SUB
_L6 = _NOFF[6]
_L9 = _NOFF[9]
_WB = 1792
_WC = 640


def _prep_body(xt_ref, w_ref):
    theta = jnp.pi / 2.0 - jnp.deg2rad(xt_ref[0:1, :])
    phi = jnp.deg2rad(xt_ref[1:2, :])
    cos_t = jnp.cos(theta)
    for l in range(_N_LEVELS):
        nside = 2 ** l
        n_ring = 4 * nside
        n_col = 3 * nside
        ring = jnp.clip(jnp.floor(theta / jnp.pi * n_ring).astype(jnp.int32),
                        0, n_ring - 1)
        col = jnp.mod(jnp.floor(phi / (2.0 * jnp.pi) * n_col).astype(jnp.int32),
                      n_col)
        nr = jnp.concatenate([jnp.clip(ring + dr, 0, n_ring - 1)
                              for dr, _ in _OFFS], axis=0)
        nc = jnp.concatenate([jnp.mod(col + dc, n_col)
                              for _, dc in _OFFS], axis=0)
        n_theta = (nr.astype(jnp.float32) + 0.5) / n_ring * jnp.pi
        n_phi = (nc.astype(jnp.float32) + 0.5) / n_col * 2.0 * jnp.pi
        dlon = n_phi - phi
        dlat = n_theta - theta
        a = (jnp.sin(dlat / 2.0) ** 2
             + cos_t * jnp.cos(n_theta) * jnp.sin(dlon / 2.0) ** 2)
        a = jnp.clip(a, 0.0, 1.0)
        dist = 2.0 * jnp.arctan2(jnp.sqrt(a), jnp.sqrt(1.0 - a))
        wgt = dist / (jnp.sum(dist, axis=0, keepdims=True) + 0.01)
        for s in range(_NSUB):
            w_ref[0, s, l * 8:(l + 1) * 8, :] = wgt[:, s * _SUB:(s + 1) * _SUB]


def _prep(xt):
    return pl.pallas_call(
        _prep_body,
        grid=(_NW,),
        in_specs=[pl.BlockSpec((2, _CHUNK), lambda i: (0, i))],
        out_specs=pl.BlockSpec((1, _NSUB, 80, _SUB), lambda i: (i, 0, 0, 0)),
        out_shape=jax.ShapeDtypeStruct((_NW, _NSUB, 80, _SUB), jnp.float32),
    )(xt)


def _indices(x):
    rad = jnp.deg2rad(x)
    theta = jnp.pi / 2.0 - rad[:, 0]
    phi = rad[:, 1]
    tiers = {"a": [], "b": [], "c": []}
    for l in range(_N_LEVELS):
        nside = 2 ** l
        n_ring = 4 * nside
        n_col = 3 * nside
        dst = tiers["a"] if l < 6 else (tiers["b"] if l < 9 else tiers["c"])
        ring = jnp.clip(jnp.floor(theta / jnp.pi * n_ring).astype(jnp.int32),
                        0, n_ring - 1)
        col = jnp.mod(jnp.floor(phi / (2.0 * jnp.pi) * n_col).astype(jnp.int32),
                      n_col)
        pix = ring * n_col + col
        dst.append(_NOFF[l] + ((pix >> 7) << 8) + (pix & 127))
        for dr, dc in _OFFS:
            nr = jnp.clip(ring + dr, 0, n_ring - 1)
            nc = jnp.mod(col + dc, n_col)
            npix = nr * n_col + nc
            dst.append(_NOFF[l] + ((npix >> 7) << 8) + (npix & 127))
    dummy = jnp.zeros((_B,), jnp.int32)
    tiers["b"].append(dummy)
    tiers["c"].append(dummy)

    def fmt(rows):
        nk = len(rows)
        return (jnp.stack(rows, axis=0)
                   .reshape(nk, _NW, _NSUB, _SUB)
                   .transpose(1, 2, 0, 3)
                   .reshape(_NW, _NSUB * nk * _SUB))

    return fmt(tiers["a"]), fmt(tiers["b"]), fmt(tiers["c"])


def _sc_body(pixa_hbm, pixb_hbm, pixc_hbm, wt_hbm, tab_hbm, out_hbm,
             s0_v, pixa_v, pixb_v, pixc_v,
             rb0_v, rb1_v, rc0_v, rc1_v, w_v, out_v, semb, semc):
    cid = lax.axis_index("c")
    sid = lax.axis_index("s")
    wid = sid * 2 + cid
    tabf1 = tab_hbm.at[pl.ds(128, _TOT - 128)]

    pltpu.sync_copy(pixc_hbm.at[wid], pixc_v)
    ccs = []
    for s8 in range(_NSUB):
        sl = pl.ds(s8 * _WC, _WC)
        ccs.append(pltpu.async_copy(tab_hbm.at[pixc_v.at[sl]],
                                    rc0_v.at[sl], semc))
        ccs.append(pltpu.async_copy(tabf1.at[pixc_v.at[sl]],
                                    rc1_v.at[sl], semc))

    pltpu.sync_copy(tab_hbm.at[pl.ds(0, _L6)], s0_v)

    pltpu.sync_copy(pixb_hbm.at[wid], pixb_v)

    cbs = []
    for s8 in range(_NSUB):
        sl = pl.ds(s8 * _WB, _WB)
        cbs.append(pltpu.async_copy(tab_hbm.at[pixb_v.at[sl]],
                                    rb0_v.at[sl], semb))
        cbs.append(pltpu.async_copy(tabf1.at[pixb_v.at[sl]],
                                    rb1_v.at[sl], semb))
    for c in ccs:
        c.wait()
    for c in cbs:
        c.wait()

    def sub_body(s, carry):
        pltpu.sync_copy(pixa_hbm.at[wid, pl.ds(s * 54 * _SUB, 54 * _SUB)],
                        pixa_v)
        pltpu.sync_copy(wt_hbm.at[wid, s], w_v)

        def group_body(g, gcarry):
            woff = s * _SUB + g * 16
            for k in range(_K):
                l, j = divmod(k, 9)
                if k < 54:
                    idx16 = pixa_v[pl.ds(k * _SUB + g * 16, 16)]
                    f0 = plsc.load_gather(s0_v, [idx16])
                    f1 = plsc.load_gather(s0_v, [idx16 + 128])
                elif k < 81:
                    ob = pl.ds(s * _WB + (k - 54) * _SUB + g * 16, 16)
                    f0 = rb0_v[ob]
                    f1 = rb1_v[ob]
                else:
                    oc = pl.ds(s * _WC + (k - 81) * _SUB + g * 16, 16)
                    f0 = rc0_v[oc]
                    f1 = rc1_v[oc]
                if j == 0:
                    acc0, acc1 = f0, f1
                else:
                    w16 = w_v[l * 8 + j - 1, pl.ds(g * 16, 16)]
                    acc0 = acc0 + w16 * f0
                    acc1 = acc1 + w16 * f1
                if j == 8:
                    out_v[2 * l, pl.ds(woff, 16)] = acc0
                    out_v[2 * l + 1, pl.ds(woff, 16)] = acc1
            return gcarry

        lax.fori_loop(0, _SUB // 16, group_body, 0)
        return carry

    lax.fori_loop(0, _NSUB, sub_body, 0)
    pltpu.sync_copy(out_v, out_hbm.at[:, pl.ds(wid * _CHUNK, _CHUNK)])


def _sc_gather(pixa, pixb, pixc, wt, tab):
    mesh = plsc.VectorSubcoreMesh(core_axis_name="c", subcore_axis_name="s")
    f = functools.partial(
        pl.kernel,
        out_type=jax.ShapeDtypeStruct((2 * _N_LEVELS, _B), jnp.float32),
        mesh=mesh,
        scratch_types=[
            pltpu.VMEM((_L6,), jnp.float32),
            pltpu.VMEM((54 * _SUB,), jnp.int32),
            pltpu.VMEM((_NSUB * _WB,), jnp.int32),
            pltpu.VMEM((_NSUB * _WC,), jnp.int32),
            pltpu.VMEM((_NSUB * _WB,), jnp.float32),
            pltpu.VMEM((_NSUB * _WB,), jnp.float32),
            pltpu.VMEM((_NSUB * _WC,), jnp.float32),
            pltpu.VMEM((_NSUB * _WC,), jnp.float32),
            pltpu.VMEM((80, _SUB), jnp.float32),
            pltpu.VMEM((2 * _N_LEVELS, _CHUNK), jnp.float32),
            pltpu.SemaphoreType.DMA,
            pltpu.SemaphoreType.DMA,
        ],
        compiler_params=pltpu.CompilerParams(needs_layout_passes=False),
    )(_sc_body)
    return f(pixa, pixb, pixc, wt, tab)


def kernel(x, tables):
    xt = x.T
    wt = _prep(xt)
    pixa, pixb, pixc = _indices(x)
    slabs = [tables[l, :_TILES[l] * 128, :]
             .reshape(_TILES[l], 128, 2)
             .transpose(0, 2, 1)
             .reshape(_TILES[l] * 256)
             for l in range(_N_LEVELS)]
    packed = jnp.concatenate(slabs)
    out2d = _sc_gather(pixa, pixb, pixc, wt, packed)
    return out2d.T


# speedup vs baseline: 83.0645x; 1.3913x over previous
import functools

import jax
import jax.numpy as jnp
from jax import lax
from jax.experimental import pallas as pl
from jax.experimental.pallas import tpu as pltpu
from jax.experimental.pallas import tpu_sc as plsc

_N_LEVELS = 10
_F = 2
_ROWS = 12 * ((2 ** (_N_LEVELS - 1)) ** 2 + 2)
_B = 16384
_OFFS = [(-1, -1), (-1, 0), (-1, 1), (0, -1), (0, 1), (1, -1), (1, 0), (1, 1)]

_USED = [12 * 4 ** l for l in range(_N_LEVELS)]
_TILES = [-(-u // 128) for u in _USED]
_NOFF = [0]
for _tl in _TILES:
    _NOFF.append(_NOFF[-1] + _tl * 256)
_TOT = _NOFF[_N_LEVELS]

_NW = 32
_CHUNK = _B // _NW
_SUB = 64
_NSUB = _CHUNK // _SUB
_K = _N_LEVELS * 9
_NIDX = _K * _SUB
_L6 = _NOFF[6]
_L9 = _NOFF[9]
_WB = 1792
_WC = 640


def _prep_body(xt_ref, w_ref):
    theta = jnp.pi / 2.0 - jnp.deg2rad(xt_ref[0:1, :])
    phi = jnp.deg2rad(xt_ref[1:2, :])
    cos_t = jnp.cos(theta)
    for l in range(_N_LEVELS):
        nside = 2 ** l
        n_ring = 4 * nside
        n_col = 3 * nside
        ring = jnp.clip(jnp.floor(theta / jnp.pi * n_ring).astype(jnp.int32),
                        0, n_ring - 1)
        col = jnp.mod(jnp.floor(phi / (2.0 * jnp.pi) * n_col).astype(jnp.int32),
                      n_col)
        nr = jnp.concatenate([jnp.clip(ring + dr, 0, n_ring - 1)
                              for dr, _ in _OFFS], axis=0)
        nc = jnp.concatenate([jnp.mod(col + dc, n_col)
                              for _, dc in _OFFS], axis=0)
        n_theta = (nr.astype(jnp.float32) + 0.5) / n_ring * jnp.pi
        n_phi = (nc.astype(jnp.float32) + 0.5) / n_col * 2.0 * jnp.pi
        dlon = n_phi - phi
        dlat = n_theta - theta
        a = (jnp.sin(dlat / 2.0) ** 2
             + cos_t * jnp.cos(n_theta) * jnp.sin(dlon / 2.0) ** 2)
        a = jnp.clip(a, 0.0, 1.0)
        dist = 2.0 * jnp.arctan2(jnp.sqrt(a), jnp.sqrt(1.0 - a))
        wgt = dist / (jnp.sum(dist, axis=0, keepdims=True) + 0.01)
        for s in range(_NSUB):
            w_ref[0, s, l * 8:(l + 1) * 8, :] = wgt[:, s * _SUB:(s + 1) * _SUB]


def _prep(xt):
    return pl.pallas_call(
        _prep_body,
        grid=(_NW,),
        in_specs=[pl.BlockSpec((2, _CHUNK), lambda i: (0, i))],
        out_specs=pl.BlockSpec((1, _NSUB, 80, _SUB), lambda i: (i, 0, 0, 0)),
        out_shape=jax.ShapeDtypeStruct((_NW, _NSUB, 80, _SUB), jnp.float32),
    )(xt)


def _indices(x):
    rad = jnp.deg2rad(x)
    theta = jnp.pi / 2.0 - rad[:, 0]
    phi = rad[:, 1]
    tiers = {"a": [], "b": [], "c": []}
    for l in range(_N_LEVELS):
        nside = 2 ** l
        n_ring = 4 * nside
        n_col = 3 * nside
        dst = tiers["a"] if l < 6 else (tiers["b"] if l < 9 else tiers["c"])
        ring = jnp.clip(jnp.floor(theta / jnp.pi * n_ring).astype(jnp.int32),
                        0, n_ring - 1)
        col = jnp.mod(jnp.floor(phi / (2.0 * jnp.pi) * n_col).astype(jnp.int32),
                      n_col)
        pix = ring * n_col + col
        dst.append(_NOFF[l] + ((pix >> 7) << 8) + (pix & 127))
        for dr, dc in _OFFS:
            nr = jnp.clip(ring + dr, 0, n_ring - 1)
            nc = jnp.mod(col + dc, n_col)
            npix = nr * n_col + nc
            dst.append(_NOFF[l] + ((npix >> 7) << 8) + (npix & 127))
    dummy = jnp.zeros((_B,), jnp.int32)
    tiers["b"].append(dummy)
    tiers["c"].append(dummy)

    def fmt(rows):
        nk = len(rows)
        return (jnp.stack(rows, axis=0)
                   .reshape(nk, _NW, _NSUB, _SUB)
                   .transpose(1, 2, 0, 3)
                   .reshape(_NW, _NSUB * nk * _SUB))

    return fmt(tiers["a"]), fmt(tiers["b"]), fmt(tiers["c"])


def _sc_body(pixa_hbm, pixb_hbm, pixc_hbm, wt_hbm, tab_hbm, out_hbm,
             s0_v, pixa_v, pixb_v, pixc_v,
             rb0_v, rb1_v, rc0_v, rc1_v, w_v, out_v, semb, semc):
    cid = lax.axis_index("c")
    sid = lax.axis_index("s")
    wid = sid * 2 + cid
    tabf1 = tab_hbm.at[pl.ds(128, _TOT + 512 - 128)]

    pltpu.sync_copy(pixc_hbm.at[wid], pixc_v)
    ccs = []
    for s8 in range(_NSUB):
        sl = pl.ds(s8 * _WC, _WC)
        ccs.append(pltpu.async_copy(tab_hbm.at[pixc_v.at[sl]],
                                    rc0_v.at[sl], semc))
        ccs.append(pltpu.async_copy(tabf1.at[pixc_v.at[sl]],
                                    rc1_v.at[sl], semc))

    pltpu.sync_copy(tab_hbm.at[pl.ds(0, _L6)], s0_v)

    pltpu.sync_copy(pixb_hbm.at[wid], pixb_v)

    cbs = []
    for s8 in range(_NSUB):
        sl = pl.ds(s8 * _WB, _WB)
        cbs.append(pltpu.async_copy(tab_hbm.at[pixb_v.at[sl]],
                                    rb0_v.at[sl], semb))
        cbs.append(pltpu.async_copy(tabf1.at[pixb_v.at[sl]],
                                    rb1_v.at[sl], semb))
    for c in ccs:
        c.wait()
    for c in cbs:
        c.wait()

    def sub_body(s, carry):
        pltpu.sync_copy(pixa_hbm.at[wid, pl.ds(s * 54 * _SUB, 54 * _SUB)],
                        pixa_v)
        pltpu.sync_copy(wt_hbm.at[wid, s], w_v)

        def group_body(g, gcarry):
            woff = s * _SUB + g * 16
            for k in range(_K):
                l, j = divmod(k, 9)
                if k < 54:
                    idx16 = pixa_v[pl.ds(k * _SUB + g * 16, 16)]
                    f0 = plsc.load_gather(s0_v, [idx16])
                    f1 = plsc.load_gather(s0_v, [idx16 + 128])
                elif k < 81:
                    ob = pl.ds(s * _WB + (k - 54) * _SUB + g * 16, 16)
                    f0 = rb0_v[ob]
                    f1 = rb1_v[ob]
                else:
                    oc = pl.ds(s * _WC + (k - 81) * _SUB + g * 16, 16)
                    f0 = rc0_v[oc]
                    f1 = rc1_v[oc]
                if j == 0:
                    acc0, acc1 = f0, f1
                else:
                    w16 = w_v[l * 8 + j - 1, pl.ds(g * 16, 16)]
                    acc0 = acc0 + w16 * f0
                    acc1 = acc1 + w16 * f1
                if j == 8:
                    out_v[2 * l, pl.ds(woff, 16)] = acc0
                    out_v[2 * l + 1, pl.ds(woff, 16)] = acc1
            return gcarry

        lax.fori_loop(0, _SUB // 16, group_body, 0)
        return carry

    lax.fori_loop(0, _NSUB, sub_body, 0)
    pltpu.sync_copy(out_v, out_hbm.at[:, pl.ds(wid * _CHUNK, _CHUNK)])


def _sc_gather(pixa, pixb, pixc, wt, tab):
    mesh = plsc.VectorSubcoreMesh(core_axis_name="c", subcore_axis_name="s")
    f = functools.partial(
        pl.kernel,
        out_type=jax.ShapeDtypeStruct((2 * _N_LEVELS, _B), jnp.float32),
        mesh=mesh,
        scratch_types=[
            pltpu.VMEM((_L6,), jnp.float32),
            pltpu.VMEM((54 * _SUB,), jnp.int32),
            pltpu.VMEM((_NSUB * _WB,), jnp.int32),
            pltpu.VMEM((_NSUB * _WC,), jnp.int32),
            pltpu.VMEM((_NSUB * _WB,), jnp.float32),
            pltpu.VMEM((_NSUB * _WB,), jnp.float32),
            pltpu.VMEM((_NSUB * _WC,), jnp.float32),
            pltpu.VMEM((_NSUB * _WC,), jnp.float32),
            pltpu.VMEM((80, _SUB), jnp.float32),
            pltpu.VMEM((2 * _N_LEVELS, _CHUNK), jnp.float32),
            pltpu.SemaphoreType.DMA,
            pltpu.SemaphoreType.DMA,
        ],
        compiler_params=pltpu.CompilerParams(needs_layout_passes=False),
    )(_sc_body)
    return f(pixa, pixb, pixc, wt, tab)


def kernel(x, tables):
    xt = x.T
    wt = _prep(xt)
    pixa, pixb, pixc = _indices(x)
    slabs = [tables[l, :_TILES[l] * 128, :]
             .reshape(_TILES[l], 128, 2)
             .transpose(0, 2, 1)
             .reshape(_TILES[l] * 2, 128)
             for l in range(_N_LEVELS)]
    slabs.append(jnp.zeros((4, 128), jnp.float32))
    packed = jnp.concatenate(slabs, axis=0).reshape(-1)
    out2d = _sc_gather(pixa, pixb, pixc, wt, packed)
    return out2d.T


# speedup vs baseline: 85.6879x; 1.0316x over previous
import functools

import jax
import jax.numpy as jnp
from jax import lax
from jax.experimental import pallas as pl
from jax.experimental.pallas import tpu as pltpu
from jax.experimental.pallas import tpu_sc as plsc

_N_LEVELS = 10
_F = 2
_ROWS = 12 * ((2 ** (_N_LEVELS - 1)) ** 2 + 2)
_B = 16384
_OFFS = [(-1, -1), (-1, 0), (-1, 1), (0, -1), (0, 1), (1, -1), (1, 0), (1, 1)]

_USED = [12 * 4 ** l for l in range(_N_LEVELS)]
_TILES = [-(-u // 128) for u in _USED]
_NOFF = [0]
for _tl in _TILES:
    _NOFF.append(_NOFF[-1] + _tl * 256)
_TOT = _NOFF[_N_LEVELS]

_NW = 32
_CHUNK = _B // _NW
_SUB = 64
_NSUB = _CHUNK // _SUB
_K = _N_LEVELS * 9
_NIDX = _K * _SUB
_L6 = _NOFF[6]
_L9 = _NOFF[9]
_WB = 1792
_WC = 640


def _prep_body(xt_ref, w_ref):
    theta = jnp.pi / 2.0 - jnp.deg2rad(xt_ref[0:1, :])
    phi = jnp.deg2rad(xt_ref[1:2, :])
    cos_t = jnp.cos(theta)
    for l in range(_N_LEVELS):
        nside = 2 ** l
        n_ring = 4 * nside
        n_col = 3 * nside
        ring = jnp.clip(jnp.floor(theta / jnp.pi * n_ring).astype(jnp.int32),
                        0, n_ring - 1)
        col = jnp.mod(jnp.floor(phi / (2.0 * jnp.pi) * n_col).astype(jnp.int32),
                      n_col)
        slat2, ccnt, slon2 = {}, {}, {}
        for d in (-1, 0, 1):
            nr = jnp.clip(ring + d, 0, n_ring - 1)
            n_theta = (nr.astype(jnp.float32) + 0.5) / n_ring * jnp.pi
            slat2[d] = jnp.sin((n_theta - theta) / 2.0) ** 2
            ccnt[d] = cos_t * jnp.cos(n_theta)
            nc = jnp.mod(col + d, n_col)
            n_phi = (nc.astype(jnp.float32) + 0.5) / n_col * 2.0 * jnp.pi
            slon2[d] = jnp.sin((n_phi - phi) / 2.0) ** 2
        a = jnp.concatenate([slat2[dr] + ccnt[dr] * slon2[dc]
                             for dr, dc in _OFFS], axis=0)
        a = jnp.clip(a, 0.0, 1.0)
        dist = 2.0 * jnp.arctan2(jnp.sqrt(a), jnp.sqrt(1.0 - a))
        wgt = dist / (jnp.sum(dist, axis=0, keepdims=True) + 0.01)
        for s in range(_NSUB):
            w_ref[0, s, l * 8:(l + 1) * 8, :] = wgt[:, s * _SUB:(s + 1) * _SUB]


def _prep(xt):
    return pl.pallas_call(
        _prep_body,
        grid=(_NW,),
        in_specs=[pl.BlockSpec((2, _CHUNK), lambda i: (0, i))],
        out_specs=pl.BlockSpec((1, _NSUB, 80, _SUB), lambda i: (i, 0, 0, 0)),
        out_shape=jax.ShapeDtypeStruct((_NW, _NSUB, 80, _SUB), jnp.float32),
    )(xt)


def _indices(x):
    rad = jnp.deg2rad(x)
    theta = jnp.pi / 2.0 - rad[:, 0]
    phi = rad[:, 1]
    tiers = {"a": [], "b": [], "c": []}
    for l in range(_N_LEVELS):
        nside = 2 ** l
        n_ring = 4 * nside
        n_col = 3 * nside
        dst = tiers["a"] if l < 6 else (tiers["b"] if l < 9 else tiers["c"])
        ring = jnp.clip(jnp.floor(theta / jnp.pi * n_ring).astype(jnp.int32),
                        0, n_ring - 1)
        col = jnp.mod(jnp.floor(phi / (2.0 * jnp.pi) * n_col).astype(jnp.int32),
                      n_col)
        pix = ring * n_col + col
        dst.append(_NOFF[l] + ((pix >> 7) << 8) + (pix & 127))
        for dr, dc in _OFFS:
            nr = jnp.clip(ring + dr, 0, n_ring - 1)
            nc = jnp.mod(col + dc, n_col)
            npix = nr * n_col + nc
            dst.append(_NOFF[l] + ((npix >> 7) << 8) + (npix & 127))
    dummy = jnp.zeros((_B,), jnp.int32)
    tiers["b"].append(dummy)
    tiers["c"].append(dummy)

    def fmt(rows):
        nk = len(rows)
        return (jnp.stack(rows, axis=0)
                   .reshape(nk, _NW, _NSUB, _SUB)
                   .transpose(1, 2, 0, 3)
                   .reshape(_NW, _NSUB * nk * _SUB))

    return fmt(tiers["a"]), fmt(tiers["b"]), fmt(tiers["c"])


def _sc_body(pixa_hbm, pixb_hbm, pixc_hbm, wt_hbm, tab_hbm, out_hbm,
             s0_v, pixa_v, pixb_v, pixc_v,
             rb0_v, rb1_v, rc0_v, rc1_v, w_v, out_v, semb, semc):
    cid = lax.axis_index("c")
    sid = lax.axis_index("s")
    wid = sid * 2 + cid
    tabf1 = tab_hbm.at[pl.ds(128, _TOT + 512 - 128)]

    pltpu.sync_copy(pixc_hbm.at[wid], pixc_v)
    ccs = []
    for s8 in range(_NSUB):
        sl = pl.ds(s8 * _WC, _WC)
        ccs.append(pltpu.async_copy(tab_hbm.at[pixc_v.at[sl]],
                                    rc0_v.at[sl], semc))
        ccs.append(pltpu.async_copy(tabf1.at[pixc_v.at[sl]],
                                    rc1_v.at[sl], semc))

    pltpu.sync_copy(tab_hbm.at[pl.ds(0, _L6)], s0_v)

    pltpu.sync_copy(pixb_hbm.at[wid], pixb_v)

    cbs = []
    for s8 in range(_NSUB):
        sl = pl.ds(s8 * _WB, _WB)
        cbs.append(pltpu.async_copy(tab_hbm.at[pixb_v.at[sl]],
                                    rb0_v.at[sl], semb))
        cbs.append(pltpu.async_copy(tabf1.at[pixb_v.at[sl]],
                                    rb1_v.at[sl], semb))
    for c in ccs:
        c.wait()
    for c in cbs:
        c.wait()

    def sub_body(s, carry):
        pltpu.sync_copy(pixa_hbm.at[wid, pl.ds(s * 54 * _SUB, 54 * _SUB)],
                        pixa_v)
        pltpu.sync_copy(wt_hbm.at[wid, s], w_v)

        def group_body(g, gcarry):
            woff = s * _SUB + g * 16
            for k in range(_K):
                l, j = divmod(k, 9)
                if k < 54:
                    idx16 = pixa_v[pl.ds(k * _SUB + g * 16, 16)]
                    f0 = plsc.load_gather(s0_v, [idx16])
                    f1 = plsc.load_gather(s0_v, [idx16 + 128])
                elif k < 81:
                    ob = pl.ds(s * _WB + (k - 54) * _SUB + g * 16, 16)
                    f0 = rb0_v[ob]
                    f1 = rb1_v[ob]
                else:
                    oc = pl.ds(s * _WC + (k - 81) * _SUB + g * 16, 16)
                    f0 = rc0_v[oc]
                    f1 = rc1_v[oc]
                if j == 0:
                    acc0, acc1 = f0, f1
                else:
                    w16 = w_v[l * 8 + j - 1, pl.ds(g * 16, 16)]
                    acc0 = acc0 + w16 * f0
                    acc1 = acc1 + w16 * f1
                if j == 8:
                    out_v[2 * l, pl.ds(woff, 16)] = acc0
                    out_v[2 * l + 1, pl.ds(woff, 16)] = acc1
            return gcarry

        lax.fori_loop(0, _SUB // 16, group_body, 0)
        return carry

    lax.fori_loop(0, _NSUB, sub_body, 0)
    pltpu.sync_copy(out_v, out_hbm.at[:, pl.ds(wid * _CHUNK, _CHUNK)])


def _sc_gather(pixa, pixb, pixc, wt, tab):
    mesh = plsc.VectorSubcoreMesh(core_axis_name="c", subcore_axis_name="s")
    f = functools.partial(
        pl.kernel,
        out_type=jax.ShapeDtypeStruct((2 * _N_LEVELS, _B), jnp.float32),
        mesh=mesh,
        scratch_types=[
            pltpu.VMEM((_L6,), jnp.float32),
            pltpu.VMEM((54 * _SUB,), jnp.int32),
            pltpu.VMEM((_NSUB * _WB,), jnp.int32),
            pltpu.VMEM((_NSUB * _WC,), jnp.int32),
            pltpu.VMEM((_NSUB * _WB,), jnp.float32),
            pltpu.VMEM((_NSUB * _WB,), jnp.float32),
            pltpu.VMEM((_NSUB * _WC,), jnp.float32),
            pltpu.VMEM((_NSUB * _WC,), jnp.float32),
            pltpu.VMEM((80, _SUB), jnp.float32),
            pltpu.VMEM((2 * _N_LEVELS, _CHUNK), jnp.float32),
            pltpu.SemaphoreType.DMA,
            pltpu.SemaphoreType.DMA,
        ],
        compiler_params=pltpu.CompilerParams(needs_layout_passes=False),
    )(_sc_body)
    return f(pixa, pixb, pixc, wt, tab)


def kernel(x, tables):
    xt = x.T
    wt = _prep(xt)
    pixa, pixb, pixc = _indices(x)
    slabs = [tables[l, :_TILES[l] * 128, :]
             .reshape(_TILES[l], 128, 2)
             .transpose(0, 2, 1)
             .reshape(_TILES[l] * 2, 128)
             for l in range(_N_LEVELS)]
    slabs.append(jnp.zeros((4, 128), jnp.float32))
    packed = jnp.concatenate(slabs, axis=0).reshape(-1)
    out2d = _sc_gather(pixa, pixb, pixc, wt, packed)
    return out2d.T
